# Initial kernel scaffold; baseline (speedup 1.0000x reference)
import jax
import jax.numpy as jnp
from jax.experimental import pallas as pl


def kernel(conv1_w, conv1_b, conv2_w, conv2_b, conv3_w, conv3_b, fc1_w, fc1_b, fc2_w, fc2_b, fc3_w, fc3_b, x_nchw):
    raise NotImplementedError("write your pallas kernel here")


# speedup vs baseline: 40.2145x; 40.2145x over previous
import jax
import jax.numpy as jnp
from jax.experimental import pallas as pl
from jax.experimental.pallas import tpu as pltpu

_TB = 32
_N_CLASSES = 62


def _parity_quads(xp, hp):
    tb = xp.shape[0]
    c = xp.shape[-1]
    h2 = hp // 2
    xr = xp.reshape(tb, h2, 2, hp, c)
    quads = {}
    for re in (0, 1):
        row = xr[:, :, re, :, :].reshape(tb, h2, h2, 2, c)
        for ce in (0, 1):
            quads[(re, ce)] = row[:, :, :, ce, :]
    return quads


def _conv_pool(x, w, b, ho, pad_hi):
    tb, h, _, cin = x.shape
    cout = w.shape[-1]
    xp = jnp.pad(x, ((0, 0), (1, pad_hi), (1, pad_hi), (0, 0)))
    quads = _parity_quads(xp, xp.shape[1])
    m = None
    for py in (0, 1):
        for px in (0, 1):
            taps = []
            for ky in range(3):
                for kx in range(3):
                    s, t = py + ky, px + kx
                    taps.append(
                        quads[(s % 2, t % 2)][:, s // 2:s // 2 + ho,
                                              t // 2:t // 2 + ho, :])
            pat = jnp.concatenate(taps, axis=-1).reshape(tb * ho * ho, 9 * cin)
            acc = jnp.dot(pat, w, preferred_element_type=jnp.float32)
            m = acc if m is None else jnp.maximum(m, acc)
    return jnp.maximum(m + b, 0.0).reshape(tb, ho, ho, cout)


def _conv1_pool(x, w, b):
    tb = x.shape[0]
    cout = w.shape[-1]
    xp = jnp.pad(x, ((0, 0), (1, 1), (1, 1)))
    xr = xp.reshape(tb, 15, 2, 30)
    quads = {}
    for re in (0, 1):
        row = xr[:, :, re, :].reshape(tb, 15, 15, 2)
        for ce in (0, 1):
            quads[(re, ce)] = row[:, :, :, ce]
    m = None
    for py in (0, 1):
        for px in (0, 1):
            taps = []
            for ky in range(3):
                for kx in range(3):
                    s, t = py + ky, px + kx
                    taps.append(
                        quads[(s % 2, t % 2)][:, s // 2:s // 2 + 14,
                                              t // 2:t // 2 + 14])
            pat = jnp.stack(taps, axis=-1).reshape(tb * 196, 9)
            acc = jnp.dot(pat, w, preferred_element_type=jnp.float32)
            m = acc if m is None else jnp.maximum(m, acc)
    return jnp.maximum(m + b, 0.0).reshape(tb, 14, 14, cout)


def _fused_kernel(x_ref, w1_ref, b1_ref, w2_ref, b2_ref, w3_ref, b3_ref,
                  f1_ref, g1_ref, f2_ref, g2_ref, f3_ref, g3_ref, o_ref):
    tb = x_ref.shape[0]
    a1 = _conv1_pool(x_ref[...], w1_ref[...], b1_ref[...])
    a2 = _conv_pool(a1, w2_ref[...], b2_ref[...], 7, 1)
    a3 = _conv_pool(a2, w3_ref[...], b3_ref[...], 3, 0)
    feats = a3.reshape(tb, 1152)
    h = jnp.dot(feats, f1_ref[...], preferred_element_type=jnp.float32)
    h = jnp.maximum(h + g1_ref[...], 0.0)
    h = jnp.dot(h, f2_ref[...], preferred_element_type=jnp.float32) + g2_ref[...]
    logits = (jnp.dot(h, f3_ref[...], preferred_element_type=jnp.float32)
              + g3_ref[...])
    mx = jnp.max(logits, axis=-1, keepdims=True)
    s = logits - mx
    lse = jnp.log(jnp.sum(jnp.exp(s), axis=-1, keepdims=True))
    o_ref[...] = s - lse


def kernel(conv1_w, conv1_b, conv2_w, conv2_b, conv3_w, conv3_b,
           fc1_w, fc1_b, fc2_w, fc2_b, fc3_w, fc3_b, x_nchw):
    n = x_nchw.shape[0]
    x = x_nchw.reshape(n, 28, 28).astype(jnp.float32)
    tb = _TB if n % _TB == 0 else 1
    out = pl.pallas_call(
        _fused_kernel,
        out_shape=jax.ShapeDtypeStruct((n, _N_CLASSES), jnp.float32),
        grid=(n // tb,),
        in_specs=[
            pl.BlockSpec((tb, 28, 28), lambda i: (i, 0, 0)),
            pl.BlockSpec(conv1_w.shape, lambda i: (0, 0)),
            pl.BlockSpec(conv1_b.shape, lambda i: (0, 0)),
            pl.BlockSpec(conv2_w.shape, lambda i: (0, 0)),
            pl.BlockSpec(conv2_b.shape, lambda i: (0, 0)),
            pl.BlockSpec(conv3_w.shape, lambda i: (0, 0)),
            pl.BlockSpec(conv3_b.shape, lambda i: (0, 0)),
            pl.BlockSpec(fc1_w.shape, lambda i: (0, 0)),
            pl.BlockSpec(fc1_b.shape, lambda i: (0, 0)),
            pl.BlockSpec(fc2_w.shape, lambda i: (0, 0)),
            pl.BlockSpec(fc2_b.shape, lambda i: (0, 0)),
            pl.BlockSpec(fc3_w.shape, lambda i: (0, 0)),
            pl.BlockSpec(fc3_b.shape, lambda i: (0, 0)),
        ],
        out_specs=pl.BlockSpec((tb, _N_CLASSES), lambda i: (i, 0)),
        compiler_params=pltpu.CompilerParams(
            dimension_semantics=("parallel",)),
    )(x, conv1_w, conv1_b, conv2_w, conv2_b, conv3_w, conv3_b,
      fc1_w, fc1_b, fc2_w, fc2_b, fc3_w, fc3_b)
    return out


# speedup vs baseline: 373.4767x; 9.2871x over previous
import jax
import jax.numpy as jnp
from jax.experimental import pallas as pl
from jax.experimental.pallas import tpu as pltpu

_TB = 128
_NCLS = 62


def _conv_frame(acc_rows, w_valid, cin):
    zcol = jnp.zeros((cin, _TB), jnp.float32)
    wp = w_valid + 2
    zrow = jnp.zeros((cin, wp * _TB), jnp.float32)
    pieces = [zrow]
    for row in acc_rows:
        pieces.append(zcol)
        pieces.append(row)
        pieces.append(zcol)
    pieces.append(zrow)
    return jnp.concatenate(pieces, axis=1)


def _conv_bias_relu(frame, w_t, h, wp, cin, cout):
    n_out = ((h - 1) * wp + h) * _TB
    acc = None
    for ky in range(3):
        for kx in range(3):
            k = ky * 3 + kx
            off = (ky * wp + kx) * _TB
            tap = frame[:, off:off + n_out]
            part = jnp.dot(w_t[k * cout:(k + 1) * cout, :], tap,
                           preferred_element_type=jnp.float32)
            acc = part if acc is None else acc + part
    return acc


def _pool_rows(m, b, h, wp, ho, wo):
    act = jnp.maximum(m + b, 0.0)
    rows = []
    for i in range(ho):
        r0 = act[:, (2 * i) * wp * _TB:((2 * i) * wp + h) * _TB]
        r1 = act[:, (2 * i + 1) * wp * _TB:((2 * i + 1) * wp + h) * _TB]
        rm = jnp.maximum(r0, r1)
        ev = jnp.concatenate(
            [rm[:, (2 * j) * _TB:(2 * j + 1) * _TB] for j in range(wo)],
            axis=1)
        od = jnp.concatenate(
            [rm[:, (2 * j + 1) * _TB:(2 * j + 2) * _TB] for j in range(wo)],
            axis=1)
        rows.append(jnp.maximum(ev, od))
    return rows


def _fused_kernel(x_ref, w1_ref, b1_ref, w2_ref, b2_ref, w3_ref, b3_ref,
                  f1_ref, g1_ref, f2_ref, g2_ref, f3_ref, g3_ref, o_ref):
    xb = x_ref[0]
    xrows = [xb[:, i * 28 * _TB:(i + 1) * 28 * _TB] for i in range(28)]
    fr1 = _conv_frame(xrows, 28, 1)
    m1 = _conv_bias_relu(fr1, w1_ref[...], 28, 30, 1, 32)
    a1 = _pool_rows(m1, b1_ref[...], 28, 30, 14, 14)
    fr2 = _conv_frame(a1, 14, 32)
    m2 = _conv_bias_relu(fr2, w2_ref[...], 14, 16, 32, 64)
    a2 = _pool_rows(m2, b2_ref[...], 14, 16, 7, 7)
    fr3 = _conv_frame(a2, 7, 64)
    m3 = _conv_bias_relu(fr3, w3_ref[...], 7, 9, 64, 128)
    a3 = _pool_rows(m3, b3_ref[...], 7, 9, 3, 3)
    feats = jnp.concatenate(
        [a3[h][:, w * _TB:(w + 1) * _TB] for h in range(3) for w in range(3)],
        axis=0)
    h1 = jnp.maximum(
        jnp.dot(f1_ref[...], feats, preferred_element_type=jnp.float32)
        + g1_ref[...], 0.0)
    h2 = (jnp.dot(f2_ref[...], h1, preferred_element_type=jnp.float32)
          + g2_ref[...])
    logits = (jnp.dot(f3_ref[...], h2, preferred_element_type=jnp.float32)
              + g3_ref[...])
    mx = jnp.max(logits, axis=0, keepdims=True)
    s = logits - mx
    lse = jnp.log(jnp.sum(jnp.exp(s), axis=0, keepdims=True))
    o_ref[...] = s - lse


def _tap_major(w, cin, cout):
    return jnp.transpose(w.reshape(9, cin, cout), (0, 2, 1)).reshape(
        9 * cout, cin)


def kernel(conv1_w, conv1_b, conv2_w, conv2_b, conv3_w, conv3_b,
           fc1_w, fc1_b, fc2_w, fc2_b, fc3_w, fc3_b, x_nchw):
    n = x_nchw.shape[0]
    nt = n // _TB
    xt = jnp.swapaxes(x_nchw.reshape(nt, _TB, 784), 1, 2).reshape(
        nt, 1, 784 * _TB).astype(jnp.float32)
    args = (
        _tap_major(conv1_w, 1, 32), jnp.transpose(conv1_b),
        _tap_major(conv2_w, 32, 64), jnp.transpose(conv2_b),
        _tap_major(conv3_w, 64, 128), jnp.transpose(conv3_b),
        jnp.transpose(fc1_w), jnp.transpose(fc1_b),
        jnp.transpose(fc2_w), jnp.transpose(fc2_b),
        jnp.transpose(fc3_w), jnp.transpose(fc3_b),
    )
    out = pl.pallas_call(
        _fused_kernel,
        out_shape=jax.ShapeDtypeStruct((_NCLS, n), jnp.float32),
        grid=(nt,),
        in_specs=[pl.BlockSpec((1, 1, 784 * _TB), lambda i: (i, 0, 0))] + [
            pl.BlockSpec(a.shape, lambda i: (0, 0)) for a in args],
        out_specs=pl.BlockSpec((_NCLS, _TB), lambda i: (0, i)),
        compiler_params=pltpu.CompilerParams(
            dimension_semantics=("parallel",)),
    )(xt, *args)
    return jnp.transpose(out)


# speedup vs baseline: 930.9611x; 2.4927x over previous
import jax
import jax.numpy as jnp
from jax.experimental import pallas as pl
from jax.experimental.pallas import tpu as pltpu

_TB = 128
_NCLS = 62


def _conv_frame(acc_rows, w_valid, cin):
    zcol = jnp.zeros((cin, _TB), jnp.float32)
    wp = w_valid + 2
    zrow = jnp.zeros((cin, wp * _TB), jnp.float32)
    pieces = [zrow]
    for row in acc_rows:
        pieces.append(zcol)
        pieces.append(row)
        pieces.append(zcol)
    pieces.append(zrow)
    return jnp.concatenate(pieces, axis=1)


def _conv_bias_relu(frame, w_t, h, wp, cin, cout):
    n_out = ((h - 1) * wp + h) * _TB
    n_row = n_out + 2 * _TB
    stack = jnp.concatenate(
        [frame[:, ky * wp * _TB:ky * wp * _TB + n_row] for ky in range(3)],
        axis=0)
    acc = None
    for kx in range(3):
        tap = stack[:, kx * _TB:kx * _TB + n_out]
        part = jnp.dot(w_t[kx * cout:(kx + 1) * cout, :], tap,
                       preferred_element_type=jnp.float32)
        acc = part if acc is None else acc + part
    return acc


def _pool_rows(m, b, h, wp, ho, wo):
    act = jnp.maximum(m + b, 0.0)
    rows = []
    for i in range(ho):
        r0 = act[:, (2 * i) * wp * _TB:((2 * i) * wp + h) * _TB]
        r1 = act[:, (2 * i + 1) * wp * _TB:((2 * i + 1) * wp + h) * _TB]
        rm = jnp.maximum(r0, r1)
        ev = jnp.concatenate(
            [rm[:, (2 * j) * _TB:(2 * j + 1) * _TB] for j in range(wo)],
            axis=1)
        od = jnp.concatenate(
            [rm[:, (2 * j + 1) * _TB:(2 * j + 2) * _TB] for j in range(wo)],
            axis=1)
        rows.append(jnp.maximum(ev, od))
    return rows


def _fused_kernel(x_ref, w1_ref, b1_ref, w2_ref, b2_ref, w3_ref, b3_ref,
                  f1_ref, g1_ref, f2_ref, g2_ref, f3_ref, g3_ref, o_ref):
    xb = x_ref[0]
    xrows = [xb[:, i * 28 * _TB:(i + 1) * 28 * _TB] for i in range(28)]
    fr1 = _conv_frame(xrows, 28, 1)
    m1 = _conv_bias_relu(fr1, w1_ref[...], 28, 30, 1, 32)
    a1 = _pool_rows(m1, b1_ref[...], 28, 30, 14, 14)
    fr2 = _conv_frame(a1, 14, 32)
    m2 = _conv_bias_relu(fr2, w2_ref[...], 14, 16, 32, 64)
    a2 = _pool_rows(m2, b2_ref[...], 14, 16, 7, 7)
    fr3 = _conv_frame(a2, 7, 64)
    m3 = _conv_bias_relu(fr3, w3_ref[...], 7, 9, 64, 128)
    a3 = _pool_rows(m3, b3_ref[...], 7, 9, 3, 3)
    feats = jnp.concatenate(
        [a3[h][:, w * _TB:(w + 1) * _TB] for h in range(3) for w in range(3)],
        axis=0)
    h1 = jnp.maximum(
        jnp.dot(f1_ref[...], feats, preferred_element_type=jnp.float32)
        + g1_ref[...], 0.0)
    h2 = (jnp.dot(f2_ref[...], h1, preferred_element_type=jnp.float32)
          + g2_ref[...])
    logits = (jnp.dot(f3_ref[...], h2, preferred_element_type=jnp.float32)
              + g3_ref[...])
    mx = jnp.max(logits, axis=0, keepdims=True)
    s = logits - mx
    lse = jnp.log(jnp.sum(jnp.exp(s), axis=0, keepdims=True))
    o_ref[...] = s - lse


def _tap_major(w, cin, cout):
    return jnp.transpose(w.reshape(3, 3, cin, cout), (1, 3, 0, 2)).reshape(
        3 * cout, 3 * cin)


def kernel(conv1_w, conv1_b, conv2_w, conv2_b, conv3_w, conv3_b,
           fc1_w, fc1_b, fc2_w, fc2_b, fc3_w, fc3_b, x_nchw):
    n = x_nchw.shape[0]
    nt = n // _TB
    xt = jnp.swapaxes(x_nchw.reshape(nt, _TB, 784), 1, 2).reshape(
        nt, 1, 784 * _TB).astype(jnp.float32)
    args = (
        _tap_major(conv1_w, 1, 32), jnp.transpose(conv1_b),
        _tap_major(conv2_w, 32, 64), jnp.transpose(conv2_b),
        _tap_major(conv3_w, 64, 128), jnp.transpose(conv3_b),
        jnp.transpose(fc1_w), jnp.transpose(fc1_b),
        jnp.transpose(fc2_w), jnp.transpose(fc2_b),
        jnp.transpose(fc3_w), jnp.transpose(fc3_b),
    )
    out = pl.pallas_call(
        _fused_kernel,
        out_shape=jax.ShapeDtypeStruct((_NCLS, n), jnp.float32),
        grid=(nt,),
        in_specs=[pl.BlockSpec((1, 1, 784 * _TB), lambda i: (i, 0, 0))] + [
            pl.BlockSpec(a.shape, lambda i: (0, 0)) for a in args],
        out_specs=pl.BlockSpec((_NCLS, _TB), lambda i: (0, i)),
        compiler_params=pltpu.CompilerParams(
            dimension_semantics=("parallel",)),
    )(xt, *args)
    return jnp.transpose(out)


# speedup vs baseline: 954.9123x; 1.0257x over previous
import jax
import jax.numpy as jnp
from jax.experimental import pallas as pl
from jax.experimental.pallas import tpu as pltpu

_TB = 128
_NCLS = 62


def _stacked_frames(rows, h, cin):
    wp = h + 2
    zc = jnp.zeros((cin, _TB), jnp.float32)
    zrow = jnp.zeros((cin, wp * _TB), jnp.float32)
    padded = ([zrow]
              + [jnp.concatenate([zc, r, zc], axis=1) for r in rows]
              + [zrow])
    return jnp.concatenate(
        [jnp.concatenate(padded[ky:ky + h], axis=1) for ky in range(3)],
        axis=0)


def _conv(stack, w_t, h, cout):
    n_out = ((h - 1) * (h + 2) + h) * _TB
    acc = None
    for kx in range(3):
        tap = stack[:, kx * _TB:kx * _TB + n_out]
        part = jnp.dot(w_t[kx * cout:(kx + 1) * cout, :], tap,
                       preferred_element_type=jnp.float32)
        acc = part if acc is None else acc + part
    return acc


def _pool_bias_relu(m, b, h, ho, wo):
    wp = h + 2
    rows = []
    for i in range(ho):
        r0 = m[:, (2 * i) * wp * _TB:((2 * i) * wp + h) * _TB]
        r1 = m[:, (2 * i + 1) * wp * _TB:((2 * i + 1) * wp + h) * _TB]
        rm = jnp.maximum(r0, r1)
        ev = jnp.concatenate(
            [rm[:, (2 * j) * _TB:(2 * j + 1) * _TB] for j in range(wo)],
            axis=1)
        od = jnp.concatenate(
            [rm[:, (2 * j + 1) * _TB:(2 * j + 2) * _TB] for j in range(wo)],
            axis=1)
        rows.append(jnp.maximum(jnp.maximum(ev, od) + b, 0.0))
    return rows


def _fused_kernel(x_ref, w1_ref, b1_ref, w2_ref, b2_ref, w3_ref, b3_ref,
                  f1_ref, g1_ref, f2_ref, g2_ref, f3_ref, g3_ref, o_ref):
    xb = x_ref[0]
    xrows = [xb[:, i * 28 * _TB:(i + 1) * 28 * _TB] for i in range(28)]
    m1 = _conv(_stacked_frames(xrows, 28, 1), w1_ref[...], 28, 32)
    a1 = _pool_bias_relu(m1, b1_ref[...], 28, 14, 14)
    m2 = _conv(_stacked_frames(a1, 14, 32), w2_ref[...], 14, 64)
    a2 = _pool_bias_relu(m2, b2_ref[...], 14, 7, 7)
    m3 = _conv(_stacked_frames(a2, 7, 64), w3_ref[...], 7, 128)
    a3 = _pool_bias_relu(m3, b3_ref[...], 7, 3, 3)
    feats = jnp.concatenate(
        [a3[h][:, w * _TB:(w + 1) * _TB] for h in range(3) for w in range(3)],
        axis=0)
    h1 = jnp.maximum(
        jnp.dot(f1_ref[...], feats, preferred_element_type=jnp.float32)
        + g1_ref[...], 0.0)
    h2 = (jnp.dot(f2_ref[...], h1, preferred_element_type=jnp.float32)
          + g2_ref[...])
    logits = (jnp.dot(f3_ref[...], h2, preferred_element_type=jnp.float32)
              + g3_ref[...])
    mx = jnp.max(logits, axis=0, keepdims=True)
    s = logits - mx
    lse = jnp.log(jnp.sum(jnp.exp(s), axis=0, keepdims=True))
    o_ref[...] = s - lse


def _tap_major(w, cin, cout):
    return jnp.transpose(w.reshape(3, 3, cin, cout), (1, 3, 0, 2)).reshape(
        3 * cout, 3 * cin)


def kernel(conv1_w, conv1_b, conv2_w, conv2_b, conv3_w, conv3_b,
           fc1_w, fc1_b, fc2_w, fc2_b, fc3_w, fc3_b, x_nchw):
    n = x_nchw.shape[0]
    nt = n // _TB
    xt = jnp.swapaxes(x_nchw.reshape(nt, _TB, 784), 1, 2).reshape(
        nt, 1, 784 * _TB).astype(jnp.float32)
    args = (
        _tap_major(conv1_w, 1, 32), jnp.transpose(conv1_b),
        _tap_major(conv2_w, 32, 64), jnp.transpose(conv2_b),
        _tap_major(conv3_w, 64, 128), jnp.transpose(conv3_b),
        jnp.transpose(fc1_w), jnp.transpose(fc1_b),
        jnp.transpose(fc2_w), jnp.transpose(fc2_b),
        jnp.transpose(fc3_w), jnp.transpose(fc3_b),
    )
    out = pl.pallas_call(
        _fused_kernel,
        out_shape=jax.ShapeDtypeStruct((_NCLS, n), jnp.float32),
        grid=(nt,),
        in_specs=[pl.BlockSpec((1, 1, 784 * _TB), lambda i: (i, 0, 0))] + [
            pl.BlockSpec(a.shape, lambda i: (0, 0)) for a in args],
        out_specs=pl.BlockSpec((_NCLS, _TB), lambda i: (0, i)),
        compiler_params=pltpu.CompilerParams(
            dimension_semantics=("parallel",)),
    )(xt, *args)
    return jnp.transpose(out)
